# baseline (speedup 1.0000x reference)
import jax
import jax.numpy as jnp
from jax.experimental import pallas as pl
from jax.experimental.pallas import tpu as pltpu

_BM = 200


def _body(alpha_ref, adj_a_ref, adj_b_ref, x_ref, w_ref, h0_ref, b_ref,
          o_ref, xw_ref):
    m = pl.program_id(0)
    bm = adj_a_ref.shape[1]

    @pl.when(m == 0)
    def _():
        xw_ref[...] = jnp.dot(x_ref[...], w_ref[...],
                              preferred_element_type=jnp.float32)

    a = alpha_ref[0]
    hi_a = jnp.dot(adj_a_ref[0], xw_ref[...],
                   preferred_element_type=jnp.float32)
    o_ref[:bm, :] = (1.0 - a) * hi_a + a * h0_ref[:bm, :] + b_ref[...]
    hi_b = jnp.dot(adj_b_ref[0], xw_ref[...],
                   preferred_element_type=jnp.float32)
    o_ref[bm:, :] = (1.0 - a) * hi_b + a * h0_ref[bm:, :] + b_ref[...]


def kernel(x, adj, h0, lamda, alpha, l, W, b):
    n, d = x.shape
    bm = _BM
    assert n % (2 * bm) == 0
    adj3 = adj.reshape(n // bm, bm, n)

    alpha_arr = jnp.reshape(alpha, (1,)).astype(jnp.float32)
    out = pl.pallas_call(
        _body,
        grid=(n // (2 * bm),),
        in_specs=[
            pl.BlockSpec(memory_space=pltpu.SMEM),
            pl.BlockSpec((1, bm, n), lambda m: (2 * m, 0, 0)),
            pl.BlockSpec((1, bm, n), lambda m: (2 * m + 1, 0, 0)),
            pl.BlockSpec((n, d), lambda m: (0, 0)),
            pl.BlockSpec((d, d), lambda m: (0, 0)),
            pl.BlockSpec((2 * bm, d), lambda m: (m, 0)),
            pl.BlockSpec((1, d), lambda m: (0, 0)),
        ],
        out_specs=pl.BlockSpec((2 * bm, d), lambda m: (m, 0)),
        out_shape=jax.ShapeDtypeStruct((n, d), jnp.float32),
        scratch_shapes=[pltpu.VMEM((n, d), jnp.float32)],
        compiler_params=pltpu.CompilerParams(
            dimension_semantics=("arbitrary",),
        ),
    )(alpha_arr, adj3, adj3, x, W, h0, b)
    return out


# speedup vs baseline: 1.0135x; 1.0135x over previous
import jax
import jax.numpy as jnp
from jax.experimental import pallas as pl
from jax.experimental.pallas import tpu as pltpu

_BM = 200


def _body(alpha_ref, adj_ref, x_ref, w_ref, h0_ref, b_ref, o_ref, xw_ref):
    m = pl.program_id(0)

    @pl.when(m == 0)
    def _():
        xw_ref[...] = jnp.dot(x_ref[...], w_ref[...],
                              preferred_element_type=jnp.float32)

    hi = jnp.dot(adj_ref[...], xw_ref[...],
                 preferred_element_type=jnp.float32)
    a = alpha_ref[0]
    o_ref[...] = (1.0 - a) * hi + a * h0_ref[...] + b_ref[...]


def kernel(x, adj, h0, lamda, alpha, l, W, b):
    n, d = x.shape
    bm = _BM if n % _BM == 0 else n

    alpha_arr = jnp.reshape(alpha, (1,)).astype(jnp.float32)
    out = pl.pallas_call(
        _body,
        grid=(n // bm,),
        in_specs=[
            pl.BlockSpec(memory_space=pltpu.SMEM),
            pl.BlockSpec((bm, n), lambda m: (m, 0)),
            pl.BlockSpec((n, d), lambda m: (0, 0)),
            pl.BlockSpec((d, d), lambda m: (0, 0)),
            pl.BlockSpec((bm, d), lambda m: (m, 0)),
            pl.BlockSpec((1, d), lambda m: (0, 0)),
        ],
        out_specs=pl.BlockSpec((bm, d), lambda m: (m, 0)),
        out_shape=jax.ShapeDtypeStruct((n, d), jnp.float32),
        scratch_shapes=[pltpu.VMEM((n, d), jnp.float32)],
        compiler_params=pltpu.CompilerParams(
            dimension_semantics=("arbitrary",),
        ),
    )(alpha_arr, adj, x, W, h0, b)
    return out


# speedup vs baseline: 1.0166x; 1.0031x over previous
import jax
import jax.numpy as jnp
from jax.experimental import pallas as pl
from jax.experimental.pallas import tpu as pltpu

_BM = 200


def _body(alpha_ref, adj_ref, x_ref, w_ref, h0_ref, b_ref, o_ref, xw_ref):
    m = pl.program_id(0)

    @pl.when(m == 0)
    def _():
        xw_ref[...] = jnp.dot(x_ref[...], w_ref[...],
                              preferred_element_type=jnp.float32)

    hi = jnp.dot(adj_ref[...], xw_ref[...],
                 preferred_element_type=jnp.float32)
    a = alpha_ref[0]
    o_ref[...] = (1.0 - a) * hi + a * h0_ref[...] + b_ref[...]


def kernel(x, adj, h0, lamda, alpha, l, W, b):
    n, d = x.shape
    bm = _BM if n % _BM == 0 else n

    alpha_arr = jnp.reshape(alpha, (1,)).astype(jnp.float32)
    out = pl.pallas_call(
        _body,
        grid=(n // bm,),
        in_specs=[
            pl.BlockSpec(memory_space=pltpu.SMEM),
            pl.BlockSpec((bm, n), lambda m: (m, 0)),
            pl.BlockSpec((n, d), lambda m: (0, 0)),
            pl.BlockSpec((d, d), lambda m: (0, 0)),
            pl.BlockSpec((bm, d), lambda m: (m, 0)),
            pl.BlockSpec((1, d), lambda m: (0, 0)),
        ],
        out_specs=pl.BlockSpec((bm, d), lambda m: (m, 0)),
        out_shape=jax.ShapeDtypeStruct((n, d), jnp.float32),
        scratch_shapes=[pltpu.VMEM((n, d), jnp.float32)],
        compiler_params=pltpu.CompilerParams(
            dimension_semantics=("parallel",),
        ),
    )(alpha_arr, adj, x, W, h0, b)
    return out
